# baseline (speedup 1.0000x reference)
import functools

import jax
import jax.numpy as jnp
from jax import lax
from jax.experimental import pallas as pl
from jax.experimental.pallas import tpu as pltpu
from jax.experimental.pallas import tpu_sc as plsc

_NUM_CORES = 2
_NUM_SUBCORES = 16
_NUM_WORKERS = _NUM_CORES * _NUM_SUBCORES
_CHUNK = 16


def _sc_gather_sum(positions, depths, seg_ids, modality_ids, node_type_ids,
                   pos_emb, depth_emb, seg_emb, modality_emb, node_type_emb):
    n = positions.shape[0]
    d = pos_emb.shape[1]
    rows_per_w = n // _NUM_WORKERS
    n_chunks = rows_per_w // _CHUNK

    mesh = plsc.VectorSubcoreMesh(core_axis_name="c", subcore_axis_name="s")

    @functools.partial(
        pl.kernel,
        out_type=jax.ShapeDtypeStruct((n, d), jnp.float32),
        mesh=mesh,
        scratch_types=(
            [pltpu.VMEM((_CHUNK,), jnp.int32) for _ in range(5)]
            + [pltpu.VMEM((_CHUNK, d), jnp.float32) for _ in range(5)]
            + [pltpu.SemaphoreType.DMA]
        ),
    )
    def gather_kernel(pos_h, dep_h, seg_h, mod_h, nty_h,
                      pe_h, de_h, se_h, me_h, ne_h,
                      out_h,
                      i0, i1, i2, i3, i4,
                      b0, b1, b2, b3, b4,
                      sem):
        wid = lax.axis_index("s") * _NUM_CORES + lax.axis_index("c")
        idx_refs = (i0, i1, i2, i3, i4)
        buf_refs = (b0, b1, b2, b3, b4)
        idx_hbm = (pos_h, dep_h, seg_h, mod_h, nty_h)
        tab_hbm = (pe_h, de_h, se_h, me_h, ne_h)
        for ch in range(n_chunks):
            base = wid * rows_per_w + ch * _CHUNK
            for iv, ih in zip(idx_refs, idx_hbm):
                pltpu.sync_copy(ih.at[pl.ds(base, _CHUNK)], iv)
            copies = [
                pltpu.async_copy(th.at[iv], bv, sem)
                for th, iv, bv in zip(tab_hbm, idx_refs, buf_refs)
            ]
            for c in copies:
                c.wait()

            def add_cols(j, _):
                col = j * 16
                for r in range(_CHUNK):
                    s = pl.ds(col, 16)
                    b0[r, s] = (b0[r, s] + b1[r, s] + b2[r, s]
                                + b3[r, s] + b4[r, s])
                return 0

            lax.fori_loop(0, d // 16, add_cols, 0, unroll=False)
            pltpu.sync_copy(b0, out_h.at[pl.ds(base, _CHUNK)])

    return gather_kernel(positions, depths, seg_ids, modality_ids,
                         node_type_ids, pos_emb, depth_emb, seg_emb,
                         modality_emb, node_type_emb)


def _tc_body(nrows, fold_from, c, eps,
             raw_ref, h_ref, bwt_ref, bb_ref, s_ref, b_ref, mw_ref, g_ref,
             o_ref):
    x = raw_ref[...]
    w = x.shape[1]
    while w > fold_from:
        half = w // 2
        x = x[:, :half] + x[:, half:w]
        w = half
    proj = lax.dot_general(x, bwt_ref[...], (((1,), (0,)), ((), ())),
                           preferred_element_type=jnp.float32)
    h = h_ref[...] + proj * (1.0 / nrows) + bb_ref[...]
    mu = jnp.mean(h, axis=-1, keepdims=True)
    xc = h - mu
    var = jnp.mean(xc * xc, axis=-1, keepdims=True)
    hn = xc * lax.rsqrt(var + eps) * s_ref[...] + b_ref[...]
    out = lax.dot_general(hn, mw_ref[...], (((1,), (1,)), ((), ())),
                          preferred_element_type=jnp.float32)
    o_ref[...] = out * g_ref[...]


def kernel(positions, depths, seg_ids, modality_ids, node_type_ids, raw_bias,
           pos_emb, depth_emb, seg_emb, modality_emb, node_type_emb,
           bias_W, bias_b, mix_W, ln_scale, ln_bias, gate):
    n, nred, c = raw_bias.shape
    d = pos_emb.shape[1]

    h_sum = _sc_gather_sum(positions, depths, seg_ids, modality_ids,
                           node_type_ids, pos_emb, depth_emb, seg_emb,
                           modality_emb, node_type_emb)

    raw2d = raw_bias.reshape(n, nred * c)
    fold_from = 128
    bwt = jnp.tile(bias_W, (fold_from // c, 1))

    rblk = 256
    grid = (n // rblk,)
    out = pl.pallas_call(
        functools.partial(_tc_body, nred, fold_from, c, 1e-5),
        grid=grid,
        in_specs=[
            pl.BlockSpec((rblk, nred * c), lambda i: (i, 0)),
            pl.BlockSpec((rblk, d), lambda i: (i, 0)),
            pl.BlockSpec((fold_from, d), lambda i: (0, 0)),
            pl.BlockSpec((1, d), lambda i: (0, 0)),
            pl.BlockSpec((1, d), lambda i: (0, 0)),
            pl.BlockSpec((1, d), lambda i: (0, 0)),
            pl.BlockSpec((d, d), lambda i: (0, 0)),
            pl.BlockSpec((1, 1), lambda i: (0, 0)),
        ],
        out_specs=pl.BlockSpec((rblk, d), lambda i: (i, 0)),
        out_shape=jax.ShapeDtypeStruct((n, d), jnp.float32),
    )(raw2d, h_sum, bwt, bias_b.reshape(1, d), ln_scale.reshape(1, d),
      ln_bias.reshape(1, d), mix_W, gate.reshape(1, 1))
    return out


# speedup vs baseline: 4.0667x; 4.0667x over previous
import functools

import jax
import jax.numpy as jnp
from jax import lax
from jax.experimental import pallas as pl
from jax.experimental.pallas import tpu as pltpu
from jax.experimental.pallas import tpu_sc as plsc

_NUM_CORES = 2
_NUM_SUBCORES = 16
_NUM_WORKERS = _NUM_CORES * _NUM_SUBCORES


def _sc_gather(indices, table):
    n = indices.shape[0]
    d = table.shape[1]
    rows_per_w = n // _NUM_WORKERS

    mesh = plsc.VectorSubcoreMesh(core_axis_name="c", subcore_axis_name="s")

    @functools.partial(
        pl.kernel,
        out_type=jax.ShapeDtypeStruct((n, d), jnp.float32),
        mesh=mesh,
        scratch_types=(
            pltpu.VMEM((rows_per_w,), jnp.int32),
            pltpu.VMEM((rows_per_w, d), jnp.float32),
            pltpu.SemaphoreType.DMA,
        ),
    )
    def gather_kernel(idx_h, tab_h, out_h, idx_v, rows_v, sem):
        wid = lax.axis_index("s") * _NUM_CORES + lax.axis_index("c")
        base = wid * rows_per_w
        pltpu.sync_copy(idx_h.at[pl.ds(base, rows_per_w)], idx_v)
        pltpu.async_copy(tab_h.at[idx_v], rows_v, sem).wait()
        pltpu.sync_copy(rows_v, out_h.at[pl.ds(base, rows_per_w)])

    return gather_kernel(indices, table)


def _tc_body(nred, eps,
             raw_ref, ep_ref, did_ref, sid_ref, mid_ref, nid_ref,
             dtab_ref, stab_ref, mtab_ref, ntab_ref,
             bw_ref, bb_ref, s_ref, b_ref, mw_ref, g_ref, o_ref):
    x3 = raw_ref[...]
    ssum = jnp.sum(x3, axis=2)
    proj = lax.dot_general(ssum, bw_ref[...], (((1,), (0,)), ((), ())),
                           preferred_element_type=jnp.float32)
    h = ep_ref[...] + proj * (1.0 / nred) + bb_ref[...]
    for id_ref, tab_ref in ((did_ref, dtab_ref), (sid_ref, stab_ref),
                            (mid_ref, mtab_ref), (nid_ref, ntab_ref)):
        v = tab_ref.shape[0]
        ids = id_ref[...]
        iot = lax.broadcasted_iota(jnp.int32, (v, ids.shape[1]), 0)
        oh = (iot == ids).astype(jnp.float32)
        h = h + lax.dot_general(oh, tab_ref[...], (((0,), (0,)), ((), ())),
                                preferred_element_type=jnp.float32)
    mu = jnp.mean(h, axis=-1, keepdims=True)
    xc = h - mu
    var = jnp.mean(xc * xc, axis=-1, keepdims=True)
    hn = xc * lax.rsqrt(var + eps) * s_ref[...] + b_ref[...]
    out = lax.dot_general(hn, mw_ref[...], (((1,), (1,)), ((), ())),
                          preferred_element_type=jnp.float32)
    o_ref[...] = out * g_ref[...]


def kernel(positions, depths, seg_ids, modality_ids, node_type_ids, raw_bias,
           pos_emb, depth_emb, seg_emb, modality_emb, node_type_emb,
           bias_W, bias_b, mix_W, ln_scale, ln_bias, gate):
    n, nred, c = raw_bias.shape
    d = pos_emb.shape[1]

    e_pos = _sc_gather(positions, pos_emb)

    raw_t = jnp.transpose(raw_bias, (0, 2, 1))

    rblk = 256
    grid = (n // rblk,)
    const = lambda i: (0, 0)
    out = pl.pallas_call(
        functools.partial(_tc_body, nred, 1e-5),
        grid=grid,
        in_specs=[
            pl.BlockSpec((rblk, c, nred), lambda i: (i, 0, 0)),
            pl.BlockSpec((rblk, d), lambda i: (i, 0)),
            pl.BlockSpec((1, rblk), lambda i: (0, i)),
            pl.BlockSpec((1, rblk), lambda i: (0, i)),
            pl.BlockSpec((1, rblk), lambda i: (0, i)),
            pl.BlockSpec((1, rblk), lambda i: (0, i)),
            pl.BlockSpec(depth_emb.shape, const),
            pl.BlockSpec(seg_emb.shape, const),
            pl.BlockSpec(modality_emb.shape, const),
            pl.BlockSpec(node_type_emb.shape, const),
            pl.BlockSpec(bias_W.shape, const),
            pl.BlockSpec((1, d), const),
            pl.BlockSpec((1, d), const),
            pl.BlockSpec((1, d), const),
            pl.BlockSpec(mix_W.shape, const),
            pl.BlockSpec((1, 1), const),
        ],
        out_specs=pl.BlockSpec((rblk, d), lambda i: (i, 0)),
        out_shape=jax.ShapeDtypeStruct((n, d), jnp.float32),
    )(raw_t, e_pos, depths.reshape(1, n), seg_ids.reshape(1, n),
      modality_ids.reshape(1, n), node_type_ids.reshape(1, n),
      depth_emb, seg_emb, modality_emb, node_type_emb,
      bias_W, bias_b.reshape(1, d), ln_scale.reshape(1, d),
      ln_bias.reshape(1, d), mix_W, gate.reshape(1, 1))
    return out
